# baseline (speedup 1.0000x reference)
import functools

import jax
import jax.numpy as jnp
from jax import lax
from jax.experimental import pallas as pl
from jax.experimental.pallas import tpu as pltpu
from jax.experimental.pallas import tpu_sc as plsc

N_NODES = 10000
N_EDGES = 320000
IN_FEATS = 128
HIDDEN = 256
NUM_CLASSES = 16

NC = 2
NS = 16
NW = NC * NS
EPW = N_EDGES // NW
K = 80
NCHUNK = EPW // K
NPAD = 10240
STRIPE = NPAD // NS

@functools.cache
def _mesh():
    return plsc.VectorSubcoreMesh(core_axis_name="c", subcore_axis_name="s")


@functools.cache
def _make_sc_degrees():
    @functools.partial(
        pl.kernel,
        mesh=_mesh(),
        out_type=jax.ShapeDtypeStruct((NC, 2, NPAD), jnp.float32),
        scratch_types=[
            pltpu.VMEM((NCHUNK, K), jnp.int32),
            pltpu.VMEM((NCHUNK, K), jnp.int32),
            pltpu.VMEM((K,), jnp.float32),
            pltpu.VMEM((STRIPE,), jnp.float32),
            pltpu.VMEM_SHARED((NPAD,), jnp.float32),
            pltpu.VMEM_SHARED((NPAD,), jnp.float32),
            pltpu.SemaphoreType.DMA,
        ],
    )
    def k(src_hbm, dst_hbm, out_hbm, idxs_v, idxd_v, ones_v, zline_v,
          dego_sh, degi_sh, sem):
        cid = lax.axis_index("c")
        sid = lax.axis_index("s")
        wid = sid * NC + cid

        def fill_ones(i, carry):
            ones_v[pl.ds(i * 16, 16)] = jnp.ones((16,), jnp.float32)
            return carry

        lax.fori_loop(0, K // 16, fill_ones, 0)

        def fill_zero(i, carry):
            zline_v[pl.ds(i * 16, 16)] = jnp.zeros((16,), jnp.float32)
            return carry

        lax.fori_loop(0, STRIPE // 16, fill_zero, 0)
        pltpu.sync_copy(src_hbm.at[wid], idxs_v)
        pltpu.sync_copy(dst_hbm.at[wid], idxd_v)
        pltpu.sync_copy(zline_v, dego_sh.at[pl.ds(sid * STRIPE, STRIPE)])
        pltpu.sync_copy(zline_v, degi_sh.at[pl.ds(sid * STRIPE, STRIPE)])
        plsc.subcore_barrier()

        def fire(j, carry):
            pltpu.async_copy(ones_v, dego_sh.at[idxs_v.at[j]], sem, add=True)
            pltpu.async_copy(ones_v, degi_sh.at[idxd_v.at[j]], sem, add=True)
            return carry

        lax.fori_loop(0, NCHUNK, fire, 0)

        def drain(j, carry):
            pltpu.make_async_copy(ones_v, dego_sh.at[idxs_v.at[j]], sem).wait()
            pltpu.make_async_copy(ones_v, degi_sh.at[idxd_v.at[j]], sem).wait()
            return carry

        lax.fori_loop(0, NCHUNK, drain, 0)
        plsc.subcore_barrier()
        pltpu.sync_copy(dego_sh.at[pl.ds(sid * STRIPE, STRIPE)],
                        out_hbm.at[cid, 0, pl.ds(sid * STRIPE, STRIPE)])
        pltpu.sync_copy(degi_sh.at[pl.ds(sid * STRIPE, STRIPE)],
                        out_hbm.at[cid, 1, pl.ds(sid * STRIPE, STRIPE)])

    return k


@functools.cache
def _make_sc_agg(d):
    nbuf = 2 if d == 128 else 4
    cp = (None if d == 128
          else pltpu.CompilerParams(use_tc_tiling_on_sc=False))

    @functools.partial(
        pl.kernel,
        mesh=_mesh(),
        out_type=jax.ShapeDtypeStruct((NC, NPAD, d), jnp.float32),
        compiler_params=cp,
        scratch_types=[
            pltpu.VMEM((EPW,), jnp.int32),
            pltpu.VMEM((NCHUNK, K), jnp.int32),
            [pltpu.VMEM((K, d), jnp.float32)] * nbuf,
            [pltpu.SemaphoreType.DMA] * nbuf,
            [pltpu.SemaphoreType.DMA] * nbuf,
            pltpu.VMEM_SHARED((NPAD, d), jnp.float32),
        ],
    )
    def k(tab2_hbm, src_flat_hbm, dst_hbm, out_hbm, idxs_v, idxd_v, rows,
          gsems, ssems, agg_sh):
        cid = lax.axis_index("c")
        sid = lax.axis_index("s")
        wid = sid * NC + cid
        r0 = rows[0]
        tab_hbm = tab2_hbm.at[cid]

        def fill_zero(i, carry):
            def inner(j, c2):
                r0[i, pl.ds(j * 16, 16)] = jnp.zeros((16,), jnp.float32)
                return c2
            return lax.fori_loop(0, d // 16, inner, carry)

        lax.fori_loop(0, K, fill_zero, 0)
        pltpu.sync_copy(src_flat_hbm.at[pl.ds(wid * EPW, EPW)], idxs_v)
        pltpu.sync_copy(dst_hbm.at[wid], idxd_v)
        for b in range(STRIPE // K):
            pltpu.sync_copy(r0, agg_sh.at[pl.ds(sid * STRIPE + b * K, K)])
        plsc.subcore_barrier()

        def gather(c, b):
            return pltpu.async_copy(
                tab_hbm.at[idxs_v.at[pl.ds(c * K, K)]], rows[b], gsems[b])

        def gather_wait(c, b):
            pltpu.make_async_copy(
                tab_hbm.at[idxs_v.at[pl.ds(c * K, K)]], rows[b],
                gsems[b]).wait()

        def scatter(c, b):
            return pltpu.async_copy(
                rows[b], agg_sh.at[idxd_v.at[c]], ssems[b], add=True)

        def scatter_wait(c, b):
            pltpu.make_async_copy(
                rows[b], agg_sh.at[idxd_v.at[c]], ssems[b]).wait()

        pltpu.sync_copy(tab_hbm.at[idxs_v.at[pl.ds((NCHUNK - 1) * K, K)]], r0)
        pltpu.sync_copy(r0, agg_sh.at[idxd_v.at[NCHUNK - 1]], add=True)

        niter = (NCHUNK - 1) // nbuf - 1
        for b in range(nbuf):
            gather(b, b)

        def body(i, carry):
            base = i * nbuf
            for b in range(nbuf):
                gather_wait(base + b, b)
                scatter(base + b, b)
            for b in range(nbuf):
                scatter_wait(base + b, b)
                gather(base + b + nbuf, b)
            return carry

        lax.fori_loop(0, niter, body, 0)
        for b in range(nbuf):
            c = niter * nbuf + b
            gather_wait(c, b)
            scatter(c, b)
        for b in range(nbuf):
            scatter_wait(niter * nbuf + b, b)
        plsc.subcore_barrier()
        pltpu.sync_copy(agg_sh.at[pl.ds(sid * STRIPE, STRIPE)],
                        out_hbm.at[cid, pl.ds(sid * STRIPE, STRIPE)])

    return k


_BLK = 1024
_GRID = NPAD // _BLK


def _norms_body(deg_ref, x_ref, xs_ref, ns_ref, nd_ref):
    dp = deg_ref[...]
    dego = dp[0, 0] + dp[1, 0]
    degi = dp[0, 1] + dp[1, 1]
    ns = lax.rsqrt(jnp.maximum(dego, 1.0))
    nd = lax.rsqrt(jnp.maximum(degi, 1.0))
    xs = x_ref[...] * ns
    xs_ref[0] = xs
    xs_ref[1] = xs
    ns_ref[...] = ns
    nd_ref[...] = nd


def _tc_norms(degp, x_pad):
    degp4 = degp.reshape(NC, 2, NPAD, 1)
    return pl.pallas_call(
        _norms_body,
        grid=(_GRID,),
        in_specs=[
            pl.BlockSpec((NC, 2, _BLK, 1), lambda i: (0, 0, i, 0)),
            pl.BlockSpec((_BLK, IN_FEATS), lambda i: (i, 0)),
        ],
        out_specs=[
            pl.BlockSpec((NC, _BLK, IN_FEATS), lambda i: (0, i, 0)),
            pl.BlockSpec((_BLK, 1), lambda i: (i, 0)),
            pl.BlockSpec((_BLK, 1), lambda i: (i, 0)),
        ],
        out_shape=[
            jax.ShapeDtypeStruct((NC, NPAD, IN_FEATS), jnp.float32),
            jax.ShapeDtypeStruct((NPAD, 1), jnp.float32),
            jax.ShapeDtypeStruct((NPAD, 1), jnp.float32),
        ],
    )(degp4, x_pad)


def _dense_body(p_ref, nd_ref, ns_ref, w1_ref, b1_ref, w2_ref, t_ref):
    a = (p_ref[0] + p_ref[1]) * nd_ref[...]
    h = jnp.dot(a, w1_ref[...], preferred_element_type=jnp.float32,
                precision=lax.Precision.HIGHEST)
    h = jnp.maximum(h + b1_ref[...], 0.0)
    t = jnp.dot(h * ns_ref[...], w2_ref[...],
                preferred_element_type=jnp.float32,
                precision=lax.Precision.HIGHEST)
    t_ref[0] = t
    t_ref[1] = t


def _tc_dense(agg1p, nd, ns, W1, b1, W2):
    return pl.pallas_call(
        _dense_body,
        grid=(_GRID,),
        in_specs=[
            pl.BlockSpec((NC, _BLK, IN_FEATS), lambda i: (0, i, 0)),
            pl.BlockSpec((_BLK, 1), lambda i: (i, 0)),
            pl.BlockSpec((_BLK, 1), lambda i: (i, 0)),
            pl.BlockSpec((IN_FEATS, HIDDEN), lambda i: (0, 0)),
            pl.BlockSpec((1, HIDDEN), lambda i: (0, 0)),
            pl.BlockSpec((HIDDEN, NUM_CLASSES), lambda i: (0, 0)),
        ],
        out_specs=pl.BlockSpec((NC, _BLK, NUM_CLASSES), lambda i: (0, i, 0)),
        out_shape=jax.ShapeDtypeStruct((NC, NPAD, NUM_CLASSES), jnp.float32),
    )(agg1p, nd, ns, W1, b1.reshape(1, HIDDEN), W2)


def _final_body(p_ref, nd_ref, b2_ref, o_ref):
    o_ref[...] = (p_ref[0] + p_ref[1]) * nd_ref[...] + b2_ref[...]


def _tc_final(agg2p, nd, b2):
    return pl.pallas_call(
        _final_body,
        grid=(_GRID,),
        in_specs=[
            pl.BlockSpec((NC, _BLK, NUM_CLASSES), lambda i: (0, i, 0)),
            pl.BlockSpec((_BLK, 1), lambda i: (i, 0)),
            pl.BlockSpec((1, NUM_CLASSES), lambda i: (0, 0)),
        ],
        out_specs=pl.BlockSpec((_BLK, NUM_CLASSES), lambda i: (i, 0)),
        out_shape=jax.ShapeDtypeStruct((NPAD, NUM_CLASSES), jnp.float32),
    )(agg2p, nd, b2.reshape(1, NUM_CLASSES))


def kernel(features, edge_index, W1, b1, att_w, att_b, W2, b2):
    del att_w, att_b
    src_flat = edge_index[0].astype(jnp.int32)
    src3 = src_flat.reshape(NW, NCHUNK, K)
    dst3 = edge_index[1].astype(jnp.int32).reshape(NW, NCHUNK, K)
    x_pad = jnp.pad(features, ((0, NPAD - N_NODES), (0, 0)))

    degp = _make_sc_degrees()(src3, dst3)
    xs, ns, nd = _tc_norms(degp, x_pad)
    agg1p = _make_sc_agg(IN_FEATS)(xs, src_flat, dst3)
    t = _tc_dense(agg1p, nd, ns, W1, b1, W2)
    agg2p = _make_sc_agg(NUM_CLASSES)(t, src_flat, dst3)
    out = _tc_final(agg2p, nd, b2)
    return out[:N_NODES]


# speedup vs baseline: 1.0119x; 1.0119x over previous
import functools

import jax
import jax.numpy as jnp
from jax import lax
from jax.experimental import pallas as pl
from jax.experimental.pallas import tpu as pltpu
from jax.experimental.pallas import tpu_sc as plsc

N_NODES = 10000
N_EDGES = 320000
IN_FEATS = 128
HIDDEN = 256
NUM_CLASSES = 16

NC = 2
NS = 16
NW = NC * NS
EPW = N_EDGES // NW
K = 80
NCHUNK = EPW // K
NPAD = 10240
STRIPE = NPAD // NS

@functools.cache
def _mesh():
    return plsc.VectorSubcoreMesh(core_axis_name="c", subcore_axis_name="s")


@functools.cache
def _make_sc_degrees():
    @functools.partial(
        pl.kernel,
        mesh=_mesh(),
        out_type=jax.ShapeDtypeStruct((NC, 2, NPAD), jnp.float32),
        scratch_types=[
            pltpu.VMEM((NCHUNK, K), jnp.int32),
            pltpu.VMEM((NCHUNK, K), jnp.int32),
            pltpu.VMEM((K,), jnp.float32),
            pltpu.VMEM((STRIPE,), jnp.float32),
            pltpu.VMEM_SHARED((NPAD,), jnp.float32),
            pltpu.VMEM_SHARED((NPAD,), jnp.float32),
            pltpu.SemaphoreType.DMA,
        ],
    )
    def k(src_hbm, dst_hbm, out_hbm, idxs_v, idxd_v, ones_v, zline_v,
          dego_sh, degi_sh, sem):
        cid = lax.axis_index("c")
        sid = lax.axis_index("s")
        wid = sid * NC + cid

        def fill_ones(i, carry):
            ones_v[pl.ds(i * 16, 16)] = jnp.ones((16,), jnp.float32)
            return carry

        lax.fori_loop(0, K // 16, fill_ones, 0)

        def fill_zero(i, carry):
            zline_v[pl.ds(i * 16, 16)] = jnp.zeros((16,), jnp.float32)
            return carry

        lax.fori_loop(0, STRIPE // 16, fill_zero, 0)
        pltpu.sync_copy(src_hbm.at[wid], idxs_v)
        pltpu.sync_copy(dst_hbm.at[wid], idxd_v)
        pltpu.sync_copy(zline_v, dego_sh.at[pl.ds(sid * STRIPE, STRIPE)])
        pltpu.sync_copy(zline_v, degi_sh.at[pl.ds(sid * STRIPE, STRIPE)])
        plsc.subcore_barrier()

        def fire(j, carry):
            pltpu.async_copy(ones_v, dego_sh.at[idxs_v.at[j]], sem, add=True)
            pltpu.async_copy(ones_v, degi_sh.at[idxd_v.at[j]], sem, add=True)
            return carry

        lax.fori_loop(0, NCHUNK, fire, 0)

        def drain(j, carry):
            pltpu.make_async_copy(ones_v, dego_sh.at[idxs_v.at[j]], sem).wait()
            pltpu.make_async_copy(ones_v, degi_sh.at[idxd_v.at[j]], sem).wait()
            return carry

        lax.fori_loop(0, NCHUNK, drain, 0)
        plsc.subcore_barrier()
        pltpu.sync_copy(dego_sh.at[pl.ds(sid * STRIPE, STRIPE)],
                        out_hbm.at[cid, 0, pl.ds(sid * STRIPE, STRIPE)])
        pltpu.sync_copy(degi_sh.at[pl.ds(sid * STRIPE, STRIPE)],
                        out_hbm.at[cid, 1, pl.ds(sid * STRIPE, STRIPE)])

    return k


@functools.cache
def _make_sc_agg(d):
    nbuf = 2 if d == 128 else 4
    cp = (None if d == 128
          else pltpu.CompilerParams(use_tc_tiling_on_sc=False))

    @functools.partial(
        pl.kernel,
        mesh=_mesh(),
        out_type=jax.ShapeDtypeStruct((NC, NPAD, d), jnp.float32),
        compiler_params=cp,
        scratch_types=[
            pltpu.VMEM((EPW,), jnp.int32),
            pltpu.VMEM((NCHUNK, K), jnp.int32),
            [pltpu.VMEM((K, d), jnp.float32)] * nbuf,
            [pltpu.SemaphoreType.DMA] * nbuf,
            [pltpu.SemaphoreType.DMA] * nbuf,
            pltpu.VMEM_SHARED((NPAD, d), jnp.float32),
        ],
    )
    def k(tab_hbm, src_flat_hbm, dst_hbm, out_hbm, idxs_v, idxd_v, rows,
          gsems, ssems, agg_sh):
        cid = lax.axis_index("c")
        sid = lax.axis_index("s")
        wid = sid * NC + cid
        r0 = rows[0]

        def fill_zero(i, carry):
            def inner(j, c2):
                r0[i, pl.ds(j * 16, 16)] = jnp.zeros((16,), jnp.float32)
                return c2
            return lax.fori_loop(0, d // 16, inner, carry)

        lax.fori_loop(0, K, fill_zero, 0)
        pltpu.sync_copy(src_flat_hbm.at[pl.ds(wid * EPW, EPW)], idxs_v)
        pltpu.sync_copy(dst_hbm.at[wid], idxd_v)
        for b in range(STRIPE // K):
            pltpu.sync_copy(r0, agg_sh.at[pl.ds(sid * STRIPE + b * K, K)])
        plsc.subcore_barrier()

        def gather(c, b):
            return pltpu.async_copy(
                tab_hbm.at[idxs_v.at[pl.ds(c * K, K)]], rows[b], gsems[b])

        def gather_wait(c, b):
            pltpu.make_async_copy(
                tab_hbm.at[idxs_v.at[pl.ds(c * K, K)]], rows[b],
                gsems[b]).wait()

        def scatter(c, b):
            return pltpu.async_copy(
                rows[b], agg_sh.at[idxd_v.at[c]], ssems[b], add=True)

        def scatter_wait(c, b):
            pltpu.make_async_copy(
                rows[b], agg_sh.at[idxd_v.at[c]], ssems[b]).wait()

        pltpu.sync_copy(tab_hbm.at[idxs_v.at[pl.ds((NCHUNK - 1) * K, K)]], r0)
        pltpu.sync_copy(r0, agg_sh.at[idxd_v.at[NCHUNK - 1]], add=True)

        niter = (NCHUNK - 1) // nbuf - 1
        for b in range(nbuf):
            gather(b, b)

        def body(i, carry):
            base = i * nbuf
            for b in range(nbuf):
                gather_wait(base + b, b)
                scatter(base + b, b)
            for b in range(nbuf):
                scatter_wait(base + b, b)
                gather(base + b + nbuf, b)
            return carry

        lax.fori_loop(0, niter, body, 0)
        for b in range(nbuf):
            c = niter * nbuf + b
            gather_wait(c, b)
            scatter(c, b)
        for b in range(nbuf):
            scatter_wait(niter * nbuf + b, b)
        plsc.subcore_barrier()
        pltpu.sync_copy(agg_sh.at[pl.ds(sid * STRIPE, STRIPE)],
                        out_hbm.at[cid, pl.ds(sid * STRIPE, STRIPE)])

    return k


_BLK = 1024
_GRID = NPAD // _BLK


def _norms_body(deg_ref, x_ref, xs_ref, ns_ref, nd_ref):
    dp = deg_ref[...]
    dego = dp[0, 0] + dp[1, 0]
    degi = dp[0, 1] + dp[1, 1]
    ns = lax.rsqrt(jnp.maximum(dego, 1.0))
    nd = lax.rsqrt(jnp.maximum(degi, 1.0))
    xs_ref[...] = x_ref[...] * ns
    ns_ref[...] = ns
    nd_ref[...] = nd


def _tc_norms(degp, x_pad):
    degp4 = degp.reshape(NC, 2, NPAD, 1)
    return pl.pallas_call(
        _norms_body,
        grid=(_GRID,),
        in_specs=[
            pl.BlockSpec((NC, 2, _BLK, 1), lambda i: (0, 0, i, 0)),
            pl.BlockSpec((_BLK, IN_FEATS), lambda i: (i, 0)),
        ],
        out_specs=[
            pl.BlockSpec((_BLK, IN_FEATS), lambda i: (i, 0)),
            pl.BlockSpec((_BLK, 1), lambda i: (i, 0)),
            pl.BlockSpec((_BLK, 1), lambda i: (i, 0)),
        ],
        out_shape=[
            jax.ShapeDtypeStruct((NPAD, IN_FEATS), jnp.float32),
            jax.ShapeDtypeStruct((NPAD, 1), jnp.float32),
            jax.ShapeDtypeStruct((NPAD, 1), jnp.float32),
        ],
    )(degp4, x_pad)


def _dense_body(p_ref, nd_ref, ns_ref, w1_ref, b1_ref, w2_ref, t_ref):
    a = (p_ref[0] + p_ref[1]) * nd_ref[...]
    h = jnp.dot(a, w1_ref[...], preferred_element_type=jnp.float32,
                precision=lax.Precision.HIGHEST)
    h = jnp.maximum(h + b1_ref[...], 0.0)
    t_ref[...] = jnp.dot(h * ns_ref[...], w2_ref[...],
                         preferred_element_type=jnp.float32,
                         precision=lax.Precision.HIGHEST)


def _tc_dense(agg1p, nd, ns, W1, b1, W2):
    return pl.pallas_call(
        _dense_body,
        grid=(_GRID,),
        in_specs=[
            pl.BlockSpec((NC, _BLK, IN_FEATS), lambda i: (0, i, 0)),
            pl.BlockSpec((_BLK, 1), lambda i: (i, 0)),
            pl.BlockSpec((_BLK, 1), lambda i: (i, 0)),
            pl.BlockSpec((IN_FEATS, HIDDEN), lambda i: (0, 0)),
            pl.BlockSpec((1, HIDDEN), lambda i: (0, 0)),
            pl.BlockSpec((HIDDEN, NUM_CLASSES), lambda i: (0, 0)),
        ],
        out_specs=pl.BlockSpec((_BLK, NUM_CLASSES), lambda i: (i, 0)),
        out_shape=jax.ShapeDtypeStruct((NPAD, NUM_CLASSES), jnp.float32),
    )(agg1p, nd, ns, W1, b1.reshape(1, HIDDEN), W2)


def _final_body(p_ref, nd_ref, b2_ref, o_ref):
    o_ref[...] = (p_ref[0] + p_ref[1]) * nd_ref[...] + b2_ref[...]


def _tc_final(agg2p, nd, b2):
    return pl.pallas_call(
        _final_body,
        grid=(_GRID,),
        in_specs=[
            pl.BlockSpec((NC, _BLK, NUM_CLASSES), lambda i: (0, i, 0)),
            pl.BlockSpec((_BLK, 1), lambda i: (i, 0)),
            pl.BlockSpec((1, NUM_CLASSES), lambda i: (0, 0)),
        ],
        out_specs=pl.BlockSpec((_BLK, NUM_CLASSES), lambda i: (i, 0)),
        out_shape=jax.ShapeDtypeStruct((NPAD, NUM_CLASSES), jnp.float32),
    )(agg2p, nd, b2.reshape(1, NUM_CLASSES))


def kernel(features, edge_index, W1, b1, att_w, att_b, W2, b2):
    del att_w, att_b
    src_flat = edge_index[0].astype(jnp.int32)
    src3 = src_flat.reshape(NW, NCHUNK, K)
    dst3 = edge_index[1].astype(jnp.int32).reshape(NW, NCHUNK, K)
    x_pad = jnp.pad(features, ((0, NPAD - N_NODES), (0, 0)))

    degp = _make_sc_degrees()(src3, dst3)
    xs, ns, nd = _tc_norms(degp, x_pad)
    agg1p = _make_sc_agg(IN_FEATS)(xs, src_flat, dst3)
    t = _tc_dense(agg1p, nd, ns, W1, b1, W2)
    agg2p = _make_sc_agg(NUM_CLASSES)(t, src_flat, dst3)
    out = _tc_final(agg2p, nd, b2)
    return out[:N_NODES]
